# Initial kernel scaffold; baseline (speedup 1.0000x reference)
import jax
import jax.numpy as jnp
from jax.experimental import pallas as pl


def kernel(x, edge_index, batch, embed_w, agent_w, W0, b0, Ws, bs, ps, Wf, bf):
    raise NotImplementedError("write your pallas kernel here")


# speedup vs baseline: 3.9088x; 3.9088x over previous
import functools
import math

import jax
import jax.numpy as jnp
from jax import lax
from jax.experimental import pallas as pl
from jax.experimental.pallas import tpu as pltpu
from jax.experimental.pallas import tpu_sc as plsc

HID = 16
N_POOL = 3
RATIO = 0.6
DIV = 1.0 / 10000.0

NC = 2
NS = 16
NW = NC * NS
CHUNK = 128
ZROWS = 128


def _pad_up(v, m):
    return ((v + m - 1) // m) * m


@functools.lru_cache(maxsize=None)
def _make_accum_kernel(nacc, epad):
    edges_per_w = epad // NW
    chunks_per_w = edges_per_w // CHUNK
    rpt = nacc // NS
    mesh = plsc.VectorSubcoreMesh(core_axis_name="c", subcore_axis_name="s")

    @functools.partial(
        pl.kernel,
        out_type=jax.ShapeDtypeStruct((NC, nacc, HID), jnp.float32),
        mesh=mesh,
        compiler_params=pltpu.CompilerParams(use_tc_tiling_on_sc=False),
        scratch_types=[
            pltpu.VMEM((ZROWS, HID), jnp.float32),
            pltpu.VMEM((CHUNK,), jnp.int32),
            pltpu.VMEM((CHUNK,), jnp.int32),
            pltpu.VMEM((CHUNK, HID), jnp.float32),
            pltpu.VMEM_SHARED((nacc, HID), jnp.float32),
            pltpu.SemaphoreType.DMA,
        ],
    )
    def accum(g_hbm, row_hbm, col_hbm, out_hbm, zero_v, ridx_v, cidx_v,
              buf_v, acc_sh, sem):
        c = lax.axis_index("c")
        s = lax.axis_index("s")
        wid = c * NS + s

        def zfill(i, _):
            zero_v[i, :] = jnp.zeros((HID,), jnp.float32)
            return _
        lax.fori_loop(0, ZROWS, zfill, None)
        for j in range(rpt // ZROWS):
            pltpu.sync_copy(zero_v, acc_sh.at[pl.ds(s * rpt + j * ZROWS, ZROWS)])
        plsc.subcore_barrier()

        base0 = wid * edges_per_w

        def body(i, _):
            base = base0 + i * CHUNK
            pltpu.sync_copy(row_hbm.at[pl.ds(base, CHUNK)], ridx_v)
            pltpu.async_copy(g_hbm.at[ridx_v], buf_v, sem).wait()
            pltpu.sync_copy(col_hbm.at[pl.ds(base, CHUNK)], cidx_v)
            pltpu.sync_copy(buf_v, acc_sh.at[cidx_v], add=True)
            return _
        lax.fori_loop(0, chunks_per_w, body, None)

        plsc.subcore_barrier()
        pltpu.sync_copy(acc_sh.at[pl.ds(s * rpt, rpt)],
                        out_hbm.at[c, pl.ds(s * rpt, rpt)])

    return accum


@functools.lru_cache(maxsize=None)
def _make_deg_kernel(nacc, epad, ndummy):
    edges_per_w = epad // NW
    chunks_per_w = edges_per_w // CHUNK
    rpt = nacc // NS
    mesh = plsc.VectorSubcoreMesh(core_axis_name="c", subcore_axis_name="s")

    @functools.partial(
        pl.kernel,
        out_type=jax.ShapeDtypeStruct((NC, nacc), jnp.float32),
        mesh=mesh,
        compiler_params=pltpu.CompilerParams(use_tc_tiling_on_sc=False),
        scratch_types=[
            pltpu.VMEM((ZROWS * HID,), jnp.float32),
            pltpu.VMEM((CHUNK,), jnp.int32),
            pltpu.VMEM((CHUNK,), jnp.int32),
            pltpu.VMEM((CHUNK,), jnp.float32),
            pltpu.VMEM_SHARED((nacc,), jnp.float32),
            pltpu.SemaphoreType.DMA,
        ],
    )
    def deg(row_hbm, col_hbm, out_hbm, zero_v, ridx_v, cidx_v, one_v,
            acc_sh, sem):
        c = lax.axis_index("c")
        s = lax.axis_index("s")
        wid = c * NS + s

        def zfill(i, _):
            zero_v[pl.ds(i * 16, 16)] = jnp.zeros((16,), jnp.float32)
            return _
        lax.fori_loop(0, ZROWS * HID // 16, zfill, None)
        zc = ZROWS * HID
        for j in range(rpt // zc):
            pltpu.sync_copy(zero_v, acc_sh.at[pl.ds(s * rpt + j * zc, zc)])
        rem = rpt % zc
        if rem:
            pltpu.sync_copy(zero_v.at[pl.ds(0, rem)],
                            acc_sh.at[pl.ds(s * rpt + (rpt // zc) * zc, rem)])
        plsc.subcore_barrier()

        base0 = wid * edges_per_w

        def body(i, _):
            base = base0 + i * CHUNK
            pltpu.sync_copy(row_hbm.at[pl.ds(base, CHUNK)], ridx_v)
            for j in range(CHUNK // 16):
                r = ridx_v[pl.ds(j * 16, 16)]
                one_v[pl.ds(j * 16, 16)] = jnp.where(
                    r == ndummy, 0.0, 1.0).astype(jnp.float32)
            pltpu.sync_copy(col_hbm.at[pl.ds(base, CHUNK)], cidx_v)
            pltpu.sync_copy(one_v, acc_sh.at[cidx_v], add=True)
            return _
        lax.fori_loop(0, chunks_per_w, body, None)

        plsc.subcore_barrier()
        pltpu.sync_copy(acc_sh.at[pl.ds(s * rpt, rpt)],
                        out_hbm.at[c, pl.ds(s * rpt, rpt)])

    return deg


def _conv(h, relu_first, W, b, row_p, col_p, dinv, nacc, n):
    hin = jax.nn.relu(h) if relu_first else h
    hw = hin @ W
    g = jnp.zeros((nacc, HID), jnp.float32).at[:n].set(hw * dinv[:, None])
    parts = _make_accum_kernel(nacc, row_p.shape[0])(g, row_p, col_p)
    acc = parts[0, :n] + parts[1, :n]
    return acc * dinv[:, None] + hw * (dinv * dinv)[:, None] + b


def kernel(x, edge_index, batch, embed_w, agent_w, W0, b0, Ws, bs, ps, Wf, bf):
    n0 = x.shape[0]
    e = edge_index.shape[1]
    epad = _pad_up(e, NW * CHUNK)

    glyphs = embed_w[x[:, 0]]
    is_agent = agent_w[x[:, 3]]
    xp = x[:, 1:2].astype(jnp.float32)
    yp = x[:, 2:3].astype(jnp.float32)
    pe_x = jnp.concatenate([jnp.sin(xp * DIV), jnp.cos(xp * DIV)], axis=0).reshape(-1, 2)
    pe_y = jnp.concatenate([jnp.sin(yp * DIV), jnp.cos(yp * DIV)], axis=0).reshape(-1, 2)
    h = jnp.concatenate([glyphs, pe_x, pe_y, is_agent], axis=-1)

    row = edge_index[0].astype(jnp.int32)
    col = edge_index[1].astype(jnp.int32)

    n = n0
    li = 0
    for pb in range(N_POOL):
        nacc = _pad_up(n + 1, NS * ZROWS)
        pad_r = jnp.full((epad - e,), n, jnp.int32)
        row_p = jnp.concatenate([row, pad_r])
        col_p = jnp.concatenate([col, pad_r])

        dparts = _make_deg_kernel(nacc, epad, n)(row_p, col_p)
        deg = dparts[0, :n] + dparts[1, :n] + 1.0
        dinv = lax.rsqrt(deg)

        if pb == 0:
            h = _conv(h, False, W0, b0, row_p, col_p, dinv, nacc, n)
        for _ in range(5):
            h = _conv(h, True, Ws[li], bs[li], row_p, col_p, dinv, nacc, n)
            li += 1

        p = ps[pb]
        score = jnp.tanh((h @ p) / jnp.linalg.norm(p))
        k = int(math.ceil(RATIO * n))
        vals, idx = lax.top_k(score, k)
        h = h[idx] * vals[:, None]
        nm = jnp.full((n + 1,), k, jnp.int32).at[idx].set(
            jnp.arange(k, dtype=jnp.int32))
        row = nm[row]
        col = nm[col]
        n = k

    pooled = jnp.mean(h, axis=0, keepdims=True)
    return pooled @ Wf + bf
